# Initial kernel scaffold; baseline (speedup 1.0000x reference)
import functools

import jax
import jax.numpy as jnp
from jax import lax
from jax.experimental import pallas as pl
from jax.experimental.pallas import tpu as pltpu
from jax.experimental.pallas import tpu_sc as plsc

N = 10000
F = 128
H = 64
L = 40
LP = 48
NNZ = 160000
E = 640000
ALPHA = 0.1
ITERS = 10

NC, NS = 2, 16
NW = NC * NS
ROWS_PER_TILE = N // NS
FPAD = 163840


def _make_sc_spmm(M, D, B):
    per_w = M // NW
    nblk = per_w // B
    ngrp = B // 16
    mesh = plsc.VectorSubcoreMesh(core_axis_name="c", subcore_axis_name="s")

    @functools.partial(
        pl.kernel,
        mesh=mesh,
        out_type=jax.ShapeDtypeStruct((NC, N, D), jnp.float32),
        scratch_types=[
            pltpu.VMEM((B,), jnp.int32),
            pltpu.VMEM((B,), jnp.int32),
            pltpu.VMEM((B,), jnp.float32),
            pltpu.VMEM((B, D), jnp.float32),
            pltpu.VMEM((ROWS_PER_TILE, D), jnp.float32),
            pltpu.VMEM_SHARED((N, D), jnp.float32),
            pltpu.SemaphoreType.DMA,
        ],
    )
    def k(src_hbm, dst_hbm, w_hbm, table_hbm, out_hbm,
          src_v, dst_v, w_v, rows_v, stage_v, acc_sh, sem):
        c = lax.axis_index("c")
        s = lax.axis_index("s")
        wid = s * NC + c

        def zrow(i, carry):
            for j in range(D // 16):
                stage_v[i, pl.ds(j * 16, 16)] = jnp.zeros((16,), jnp.float32)
            return carry
        lax.fori_loop(0, ROWS_PER_TILE, zrow, 0)
        pltpu.sync_copy(stage_v, acc_sh.at[pl.ds(s * ROWS_PER_TILE, ROWS_PER_TILE)])
        plsc.subcore_barrier()

        base = wid * per_w

        def blk(b, carry):
            off = base + b * B
            pltpu.sync_copy(src_hbm.at[pl.ds(off, B)], src_v)
            pltpu.sync_copy(dst_hbm.at[pl.ds(off, B)], dst_v)
            pltpu.sync_copy(w_hbm.at[pl.ds(off, B)], w_v)
            pltpu.async_copy(table_hbm.at[src_v], rows_v, sem).wait()
            for g in range(ngrp):
                wv = w_v[pl.ds(g * 16, 16)]
                row_ids = lax.iota(jnp.int32, 16) + g * 16
                for j in range(D):
                    col_ids = jnp.full((16,), j, jnp.int32)
                    vals = plsc.load_gather(rows_v, [row_ids, col_ids])
                    plsc.store_scatter(rows_v, [row_ids, col_ids], vals * wv)
            pltpu.sync_copy(rows_v, acc_sh.at[dst_v], add=True)
            return carry
        lax.fori_loop(0, nblk, blk, 0)
        plsc.subcore_barrier()

        pltpu.sync_copy(acc_sh.at[pl.ds(s * ROWS_PER_TILE, ROWS_PER_TILE)], stage_v)
        pltpu.sync_copy(stage_v, out_hbm.at[c, pl.ds(s * ROWS_PER_TILE, ROWS_PER_TILE)])

    return k


_fspmm = _make_sc_spmm(FPAD, H, 80)
_pspmm = _make_sc_spmm(E, LP, 80)


_R = 2500


def _dense_body(p_ref, b1_ref, w2_ref, b2_ref, out_ref):
    l1 = jnp.maximum(p_ref[0] + p_ref[1] + b1_ref[...], 0.0)
    out_ref[...] = (
        jnp.dot(l1, w2_ref[...], preferred_element_type=jnp.float32) + b2_ref[...]
    )


def _dense(p, b1, w2p, b2p):
    return pl.pallas_call(
        _dense_body,
        grid=(N // _R,),
        in_specs=[
            pl.BlockSpec((NC, _R, H), lambda i: (0, i, 0)),
            pl.BlockSpec((1, H), lambda i: (0, 0)),
            pl.BlockSpec((H, LP), lambda i: (0, 0)),
            pl.BlockSpec((1, LP), lambda i: (0, 0)),
        ],
        out_specs=pl.BlockSpec((_R, LP), lambda i: (i, 0)),
        out_shape=jax.ShapeDtypeStruct((N, LP), jnp.float32),
    )(p, b1, w2p, b2p)


def _comb_body(q_ref, l2_ref, out_ref):
    out_ref[...] = (1.0 - ALPHA) * (q_ref[0] + q_ref[1]) + ALPHA * l2_ref[...]


def _combine(q, l2):
    return pl.pallas_call(
        _comb_body,
        grid=(N // _R,),
        in_specs=[
            pl.BlockSpec((NC, _R, LP), lambda i: (0, i, 0)),
            pl.BlockSpec((_R, LP), lambda i: (i, 0)),
        ],
        out_specs=pl.BlockSpec((_R, LP), lambda i: (i, 0)),
        out_shape=jax.ShapeDtypeStruct((N, LP), jnp.float32),
    )(q, l2)


def _lsm_body(x_ref, out_ref):
    x = x_ref[...][:, :L]
    m = jnp.max(x, axis=1, keepdims=True)
    e = jnp.exp(x - m)
    lse = jnp.log(jnp.sum(e, axis=1, keepdims=True)) + m
    out_ref[...] = x - lse


def _lsm(x):
    return pl.pallas_call(
        _lsm_body,
        grid=(N // _R,),
        in_specs=[pl.BlockSpec((_R, LP), lambda i: (i, 0))],
        out_specs=pl.BlockSpec((_R, L), lambda i: (i, 0)),
        out_shape=jax.ShapeDtypeStruct((N, L), jnp.float32),
    )(x)


def kernel(feature_indices, feature_values, edge_indices, edge_weights, W1, b1, W2, b2):
    f_rows = feature_indices[0]
    f_cols = feature_indices[1]
    e_dst = edge_indices[0]
    e_src = edge_indices[1]

    pad = FPAD - NNZ
    zi = jnp.zeros((pad,), jnp.int32)
    f_rows_p = jnp.concatenate([f_rows, zi])
    f_cols_p = jnp.concatenate([f_cols, zi])
    f_vals_p = jnp.concatenate([feature_values, jnp.zeros((pad,), jnp.float32)])

    p = _fspmm(f_cols_p, f_rows_p, f_vals_p, W1)

    w2p = jnp.pad(W2, ((0, 0), (0, LP - L)))
    b2p = jnp.pad(b2, (0, LP - L)).reshape(1, LP)
    l2 = _dense(p, b1.reshape(1, H), w2p, b2p)

    loc = l2
    for _ in range(ITERS):
        q = _pspmm(e_src, e_dst, edge_weights, loc)
        loc = _combine(q, l2)
    return _lsm(loc)


# speedup vs baseline: 6.1165x; 6.1165x over previous
import functools

import jax
import jax.numpy as jnp
from jax import lax
from jax.experimental import pallas as pl
from jax.experimental.pallas import tpu as pltpu
from jax.experimental.pallas import tpu_sc as plsc

N = 10000
F = 128
H = 64
L = 40
LP = 48
NNZ = 160000
E = 640000
ALPHA = 0.1
ITERS = 10

NC, NS = 2, 16
NW = NC * NS
ROWS_PER_TILE = N // NS
FPAD = 163840


def _make_sc_spmm(M, D, B):
    per_w = M // NW
    nblk = per_w // B
    ngrp = B // 16
    mesh = plsc.VectorSubcoreMesh(core_axis_name="c", subcore_axis_name="s")

    @functools.partial(
        pl.kernel,
        mesh=mesh,
        compiler_params=pltpu.CompilerParams(use_tc_tiling_on_sc=False),
        out_type=jax.ShapeDtypeStruct((NC, N, D), jnp.float32),
        scratch_types=[
            pltpu.VMEM((B,), jnp.int32),
            pltpu.VMEM((B,), jnp.int32),
            pltpu.VMEM((B,), jnp.float32),
            pltpu.VMEM((B, D), jnp.float32),
            pltpu.VMEM((ROWS_PER_TILE, D), jnp.float32),
            pltpu.VMEM_SHARED((N, D), jnp.float32),
            pltpu.SemaphoreType.DMA,
        ],
    )
    def k(src_hbm, dst_hbm, w_hbm, table_hbm, out_hbm,
          src_v, dst_v, w_v, rows_v, stage_v, acc_sh, sem):
        c = lax.axis_index("c")
        s = lax.axis_index("s")
        wid = s * NC + c

        def zrow(i, carry):
            for j in range(D // 16):
                stage_v[i, pl.ds(j * 16, 16)] = jnp.zeros((16,), jnp.float32)
            return carry
        lax.fori_loop(0, ROWS_PER_TILE, zrow, 0)
        pltpu.sync_copy(stage_v, acc_sh.at[pl.ds(s * ROWS_PER_TILE, ROWS_PER_TILE)])
        plsc.subcore_barrier()

        base = wid * per_w

        def blk(b, carry):
            off = base + b * B
            pltpu.sync_copy(src_hbm.at[pl.ds(off, B)], src_v)
            pltpu.sync_copy(dst_hbm.at[pl.ds(off, B)], dst_v)
            pltpu.sync_copy(w_hbm.at[pl.ds(off, B)], w_v)
            pltpu.async_copy(table_hbm.at[src_v], rows_v, sem).wait()
            for g in range(ngrp):
                w16 = w_v[pl.ds(g * 16, 16)]
                for e in range(16):
                    wb = w16.at[jnp.full((16,), e, jnp.int32)].get(
                        mode="promise_in_bounds")
                    r = g * 16 + e
                    for j in range(D // 16):
                        sl = pl.ds(j * 16, 16)
                        rows_v[r, sl] = rows_v[r, sl] * wb
            pltpu.sync_copy(rows_v, acc_sh.at[dst_v], add=True)
            return carry
        lax.fori_loop(0, nblk, blk, 0)
        plsc.subcore_barrier()

        pltpu.sync_copy(acc_sh.at[pl.ds(s * ROWS_PER_TILE, ROWS_PER_TILE)], stage_v)
        pltpu.sync_copy(stage_v, out_hbm.at[c, pl.ds(s * ROWS_PER_TILE, ROWS_PER_TILE)])

    return k


_fspmm = _make_sc_spmm(FPAD, H, 80)
_pspmm = _make_sc_spmm(E, LP, 80)


_R = 2000


def _dense_body(p_ref, b1_ref, w2_ref, b2_ref, out_ref):
    l1 = jnp.maximum(p_ref[0] + p_ref[1] + b1_ref[...], 0.0)
    out_ref[...] = (
        jnp.dot(l1, w2_ref[...], preferred_element_type=jnp.float32) + b2_ref[...]
    )


def _dense(p, b1, w2p, b2p):
    return pl.pallas_call(
        _dense_body,
        grid=(N // _R,),
        in_specs=[
            pl.BlockSpec((NC, _R, H), lambda i: (0, i, 0)),
            pl.BlockSpec((1, H), lambda i: (0, 0)),
            pl.BlockSpec((H, LP), lambda i: (0, 0)),
            pl.BlockSpec((1, LP), lambda i: (0, 0)),
        ],
        out_specs=pl.BlockSpec((_R, LP), lambda i: (i, 0)),
        out_shape=jax.ShapeDtypeStruct((N, LP), jnp.float32),
    )(p, b1, w2p, b2p)


def _comb_body(q_ref, l2_ref, out_ref):
    out_ref[...] = (1.0 - ALPHA) * (q_ref[0] + q_ref[1]) + ALPHA * l2_ref[...]


def _combine(q, l2):
    return pl.pallas_call(
        _comb_body,
        grid=(N // _R,),
        in_specs=[
            pl.BlockSpec((NC, _R, LP), lambda i: (0, i, 0)),
            pl.BlockSpec((_R, LP), lambda i: (i, 0)),
        ],
        out_specs=pl.BlockSpec((_R, LP), lambda i: (i, 0)),
        out_shape=jax.ShapeDtypeStruct((N, LP), jnp.float32),
    )(q, l2)


def _lsm_body(x_ref, out_ref):
    x = x_ref[...][:, :L]
    m = jnp.max(x, axis=1, keepdims=True)
    e = jnp.exp(x - m)
    lse = jnp.log(jnp.sum(e, axis=1, keepdims=True)) + m
    out_ref[...] = x - lse


def _lsm(x):
    return pl.pallas_call(
        _lsm_body,
        grid=(N // _R,),
        in_specs=[pl.BlockSpec((_R, LP), lambda i: (i, 0))],
        out_specs=pl.BlockSpec((_R, L), lambda i: (i, 0)),
        out_shape=jax.ShapeDtypeStruct((N, L), jnp.float32),
    )(x)


def kernel(feature_indices, feature_values, edge_indices, edge_weights, W1, b1, W2, b2):
    f_rows = feature_indices[0]
    f_cols = feature_indices[1]
    e_dst = edge_indices[0]
    e_src = edge_indices[1]

    pad = FPAD - NNZ
    zi = jnp.zeros((pad,), jnp.int32)
    f_rows_p = jnp.concatenate([f_rows, zi])
    f_cols_p = jnp.concatenate([f_cols, zi])
    f_vals_p = jnp.concatenate([feature_values, jnp.zeros((pad,), jnp.float32)])

    p = _fspmm(f_cols_p, f_rows_p, f_vals_p, W1)

    w2p = jnp.pad(W2, ((0, 0), (0, LP - L)))
    b2p = jnp.pad(b2, (0, LP - L)).reshape(1, LP)
    l2 = _dense(p, b1.reshape(1, H), w2p, b2p)

    loc = l2
    for _ in range(ITERS):
        q = _pspmm(e_src, e_dst, edge_weights, loc)
        loc = _combine(q, l2)
    return _lsm(loc)


# speedup vs baseline: 10.8828x; 1.7793x over previous
import functools

import jax
import jax.numpy as jnp
from jax import lax
from jax.experimental import pallas as pl
from jax.experimental.pallas import tpu as pltpu
from jax.experimental.pallas import tpu_sc as plsc

N = 10000
F = 128
H = 64
L = 40
LP = 48
NNZ = 160000
E = 640000
ALPHA = 0.1
ITERS = 10

NC, NS = 2, 16
NW = NC * NS
RT = N // NS
SB = 512
KB = SB // 128
FPAD = 163840
EPAD = 655360


def _wbcast(w16, e):
    return w16.at[jnp.full((16,), e, jnp.int32)].get(mode="promise_in_bounds")


def _make_sc_spmm(M, D):
    per_w = M // NW
    rpw = per_w // 128
    nsb = per_w // SB
    mesh = plsc.VectorSubcoreMesh(core_axis_name="c", subcore_axis_name="s")

    @functools.partial(
        pl.kernel,
        mesh=mesh,
        compiler_params=pltpu.CompilerParams(use_tc_tiling_on_sc=False),
        out_type=pltpu.HBM((NC, N, D), jnp.float32),
        scratch_types=[
            pltpu.VMEM((2, KB, 128), jnp.int32),
            pltpu.VMEM((2, KB, 128), jnp.int32),
            pltpu.VMEM((2, KB, 128), jnp.float32),
            pltpu.VMEM((2, KB, 128, D), jnp.float32),
            pltpu.VMEM((RT // 5, D), jnp.float32),
            pltpu.VMEM_SHARED((N, D), jnp.float32),
            pltpu.SemaphoreType.DMA((2,)),
            pltpu.SemaphoreType.DMA((2,)),
            pltpu.SemaphoreType.DMA,
        ],
    )
    def k(src_hbm, dst_hbm, w_hbm, table_hbm, q_out,
          src_v, dst_v, w_v, rows_v, a_v, acc_sh, sem_i, sem_g, sem_s):
        c = lax.axis_index("c")
        s = lax.axis_index("s")
        wid = s * NC + c
        RC = RT // 5

        def zrow(i, carry):
            for j in range(D // 16):
                a_v[i, pl.ds(j * 16, 16)] = jnp.zeros((16,), jnp.float32)
            return carry
        lax.fori_loop(0, RC, zrow, 0)
        for t in range(5):
            pltpu.sync_copy(a_v, acc_sh.at[pl.ds(s * RT + t * RC, RC)])
        plsc.subcore_barrier()

        base = wid * rpw

        def stage_issue(sb, slot):
            row0 = base + sb * KB
            pltpu.async_copy(src_hbm.at[pl.ds(row0, KB)], src_v.at[slot], sem_i.at[slot])
            pltpu.async_copy(dst_hbm.at[pl.ds(row0, KB)], dst_v.at[slot], sem_i.at[slot])
            pltpu.async_copy(w_hbm.at[pl.ds(row0, KB)], w_v.at[slot], sem_i.at[slot])

        def stage_wait(sb, slot):
            row0 = base + sb * KB
            pltpu.make_async_copy(src_hbm.at[pl.ds(row0, KB)], src_v.at[slot], sem_i.at[slot]).wait()
            pltpu.make_async_copy(dst_hbm.at[pl.ds(row0, KB)], dst_v.at[slot], sem_i.at[slot]).wait()
            pltpu.make_async_copy(w_hbm.at[pl.ds(row0, KB)], w_v.at[slot], sem_i.at[slot]).wait()

        def gathers_issue(slot):
            for k2 in range(KB):
                pltpu.async_copy(table_hbm.at[src_v.at[slot, k2]], rows_v.at[slot, k2], sem_g.at[slot])

        def gathers_wait(slot):
            for k2 in range(KB):
                pltpu.make_async_copy(table_hbm.at[src_v.at[slot, k2]], rows_v.at[slot, k2], sem_g.at[slot]).wait()

        stage_issue(0, 0)
        stage_wait(0, 0)
        gathers_issue(0)
        stage_issue(1, 1)

        def sb_step(sb, carry):
            pb = sb % 2
            gathers_wait(pb)

            @pl.when(sb + 1 < nsb)
            def _prefetch():
                stage_wait(sb + 1, 1 - pb)
                gathers_issue(1 - pb)

            def panel(k2, carry2):
                def grp(g, carry3):
                    w16 = w_v[pb, k2, pl.ds(g * 16, 16)]
                    for e in range(16):
                        wb = _wbcast(w16, e)
                        for j in range(D // 16):
                            sl = pl.ds(j * 16, 16)
                            rows_v[pb, k2, g * 16 + e, sl] = rows_v[pb, k2, g * 16 + e, sl] * wb
                    return carry3
                return lax.fori_loop(0, 8, grp, carry2)
            lax.fori_loop(0, KB, panel, 0)

            hs = [pltpu.async_copy(rows_v.at[pb, k2], acc_sh.at[dst_v.at[pb, k2]], sem_s, add=True)
                  for k2 in range(KB)]
            for h in hs:
                h.wait()

            @pl.when(sb + 2 < nsb)
            def _stage_next():
                stage_issue(sb + 2, pb)
            return carry
        lax.fori_loop(0, nsb, sb_step, 0)
        plsc.subcore_barrier()

        for t in range(5):
            rs = pl.ds(s * RT + t * RC, RC)
            pltpu.sync_copy(acc_sh.at[rs], a_v)
            pltpu.sync_copy(a_v, q_out.at[c, rs])

    return k


_fspmm = _make_sc_spmm(FPAD, H)
_pspmm = _make_sc_spmm(EPAD, LP)


_R = 2000


def _dense_body(p_ref, b1_ref, w2_ref, b2_ref, out_ref):
    l1 = jnp.maximum(p_ref[0] + p_ref[1] + b1_ref[...], 0.0)
    out_ref[...] = (
        jnp.dot(l1, w2_ref[...], preferred_element_type=jnp.float32) + b2_ref[...]
    )


def _dense(p, b1, w2p, b2p):
    return pl.pallas_call(
        _dense_body,
        grid=(N // _R,),
        in_specs=[
            pl.BlockSpec((NC, _R, H), lambda i: (0, i, 0)),
            pl.BlockSpec((1, H), lambda i: (0, 0)),
            pl.BlockSpec((H, LP), lambda i: (0, 0)),
            pl.BlockSpec((1, LP), lambda i: (0, 0)),
        ],
        out_specs=pl.BlockSpec((_R, LP), lambda i: (i, 0)),
        out_shape=jax.ShapeDtypeStruct((N, LP), jnp.float32),
    )(p, b1, w2p, b2p)


def _comb_body(q_ref, l2_ref, out_ref):
    out_ref[...] = (1.0 - ALPHA) * (q_ref[0] + q_ref[1]) + ALPHA * l2_ref[...]


def _combine(q, l2):
    return pl.pallas_call(
        _comb_body,
        grid=(N // _R,),
        in_specs=[
            pl.BlockSpec((NC, _R, LP), lambda i: (0, i, 0)),
            pl.BlockSpec((_R, LP), lambda i: (i, 0)),
        ],
        out_specs=pl.BlockSpec((_R, LP), lambda i: (i, 0)),
        out_shape=jax.ShapeDtypeStruct((N, LP), jnp.float32),
    )(q, l2)


def _lsm_body(q_ref, l2_ref, out_ref):
    x = (1.0 - ALPHA) * (q_ref[0] + q_ref[1]) + ALPHA * l2_ref[...]
    x = x[:, :L]
    m = jnp.max(x, axis=1, keepdims=True)
    e = jnp.exp(x - m)
    lse = jnp.log(jnp.sum(e, axis=1, keepdims=True)) + m
    out_ref[...] = x - lse


def _lsm(q, l2):
    return pl.pallas_call(
        _lsm_body,
        grid=(N // _R,),
        in_specs=[
            pl.BlockSpec((NC, _R, LP), lambda i: (0, i, 0)),
            pl.BlockSpec((_R, LP), lambda i: (i, 0)),
        ],
        out_specs=pl.BlockSpec((_R, L), lambda i: (i, 0)),
        out_shape=jax.ShapeDtypeStruct((N, L), jnp.float32),
    )(q, l2)


def _pad2d(x, m, dtype):
    return jnp.concatenate([x, jnp.zeros((m - x.shape[0],), dtype)]).reshape(-1, 128)


def kernel(feature_indices, feature_values, edge_indices, edge_weights, W1, b1, W2, b2):
    f_src = _pad2d(feature_indices[1], FPAD, jnp.int32)
    f_dst = _pad2d(feature_indices[0], FPAD, jnp.int32)
    f_w = _pad2d(feature_values, FPAD, jnp.float32)
    e_src = _pad2d(edge_indices[1], EPAD, jnp.int32)
    e_dst = _pad2d(edge_indices[0], EPAD, jnp.int32)
    e_w = _pad2d(edge_weights, EPAD, jnp.float32)

    p = _fspmm(f_src, f_dst, f_w, W1)

    w2p = jnp.pad(W2, ((0, 0), (0, LP - L)))
    b2p = jnp.pad(b2, (0, LP - L)).reshape(1, LP)
    l2 = _dense(p, b1.reshape(1, H), w2p, b2p)

    loc = l2
    for i in range(ITERS):
        q = _pspmm(e_src, e_dst, e_w, loc)
        if i + 1 < ITERS:
            loc = _combine(q, l2)
    return _lsm(q, l2)


# speedup vs baseline: 22.1749x; 2.0376x over previous
import functools

import jax
import jax.numpy as jnp
from jax import lax
from jax.experimental import pallas as pl
from jax.experimental.pallas import tpu as pltpu
from jax.experimental.pallas import tpu_sc as plsc

N = 10000
F = 128
H = 64
L = 40
LP = 48
NNZ = 160000
E = 640000
ALPHA = 0.1
ITERS = 10

NC, NS = 2, 16
NW = NC * NS
RT = N // NS
SB = 512
KB = SB // 128
FPAD = 163840
EPAD = 655360


def _wbcast(w16, e):
    return w16.at[jnp.full((16,), e, jnp.int32)].get(mode="promise_in_bounds")


def _make_sc_spmm(M, D, T):
    per_w = M // NW
    rpw = per_w // 128
    nsb = per_w // SB
    tpt = T // NS
    mesh = plsc.VectorSubcoreMesh(core_axis_name="c", subcore_axis_name="s")

    @functools.partial(
        pl.kernel,
        mesh=mesh,
        compiler_params=pltpu.CompilerParams(use_tc_tiling_on_sc=False),
        out_type=pltpu.HBM((NC, N, D), jnp.float32),
        scratch_types=[
            pltpu.VMEM((2, KB, 128), jnp.int32),
            pltpu.VMEM((2, KB, 128), jnp.int32),
            pltpu.VMEM((2, KB, 128), jnp.float32),
            pltpu.VMEM((2, KB, 128, D), jnp.float32),
            pltpu.VMEM((RT // 5, D), jnp.float32),
            pltpu.VMEM_SHARED((N, D), jnp.float32),
            pltpu.VMEM_SHARED((T, D), jnp.float32),
            pltpu.SemaphoreType.DMA((2,)),
            pltpu.SemaphoreType.DMA((2,)),
            pltpu.SemaphoreType.DMA,
        ],
    )
    def k(src_hbm, dst_hbm, w_hbm, table_hbm, q_out,
          src_v, dst_v, w_v, rows_v, a_v, acc_sh, tab_sh, sem_i, sem_g, sem_s):
        c = lax.axis_index("c")
        s = lax.axis_index("s")
        wid = s * NC + c
        RC = RT // 5

        def zrow(i, carry):
            for j in range(D // 16):
                a_v[i, pl.ds(j * 16, 16)] = jnp.zeros((16,), jnp.float32)
            return carry
        lax.fori_loop(0, RC, zrow, 0)
        for t in range(5):
            pltpu.sync_copy(a_v, acc_sh.at[pl.ds(s * RT + t * RC, RC)])
        pltpu.sync_copy(table_hbm.at[pl.ds(s * tpt, tpt)], tab_sh.at[pl.ds(s * tpt, tpt)])
        plsc.subcore_barrier()

        base = wid * rpw

        def stage_issue(sb, slot):
            row0 = base + sb * KB
            pltpu.async_copy(src_hbm.at[pl.ds(row0, KB)], src_v.at[slot], sem_i.at[slot])
            pltpu.async_copy(dst_hbm.at[pl.ds(row0, KB)], dst_v.at[slot], sem_i.at[slot])
            pltpu.async_copy(w_hbm.at[pl.ds(row0, KB)], w_v.at[slot], sem_i.at[slot])

        def stage_wait(sb, slot):
            row0 = base + sb * KB
            pltpu.make_async_copy(src_hbm.at[pl.ds(row0, KB)], src_v.at[slot], sem_i.at[slot]).wait()
            pltpu.make_async_copy(dst_hbm.at[pl.ds(row0, KB)], dst_v.at[slot], sem_i.at[slot]).wait()
            pltpu.make_async_copy(w_hbm.at[pl.ds(row0, KB)], w_v.at[slot], sem_i.at[slot]).wait()

        def gathers_issue(slot):
            for k2 in range(KB):
                pltpu.async_copy(tab_sh.at[src_v.at[slot, k2]], rows_v.at[slot, k2], sem_g.at[slot])

        def gathers_wait(slot):
            for k2 in range(KB):
                pltpu.make_async_copy(tab_sh.at[src_v.at[slot, k2]], rows_v.at[slot, k2], sem_g.at[slot]).wait()

        stage_issue(0, 0)
        stage_wait(0, 0)
        gathers_issue(0)
        stage_issue(1, 1)

        def sb_step(sb, carry):
            pb = sb % 2
            gathers_wait(pb)

            @pl.when(sb + 1 < nsb)
            def _prefetch():
                stage_wait(sb + 1, 1 - pb)
                gathers_issue(1 - pb)

            def panel(k2, carry2):
                def grp(g, carry3):
                    w16 = w_v[pb, k2, pl.ds(g * 16, 16)]
                    for e in range(16):
                        wb = _wbcast(w16, e)
                        for j in range(D // 16):
                            sl = pl.ds(j * 16, 16)
                            rows_v[pb, k2, g * 16 + e, sl] = rows_v[pb, k2, g * 16 + e, sl] * wb
                    return carry3
                return lax.fori_loop(0, 8, grp, carry2)
            lax.fori_loop(0, KB, panel, 0)

            hs = [pltpu.async_copy(rows_v.at[pb, k2], acc_sh.at[dst_v.at[pb, k2]], sem_s, add=True)
                  for k2 in range(KB)]
            for h in hs:
                h.wait()

            @pl.when(sb + 2 < nsb)
            def _stage_next():
                stage_issue(sb + 2, pb)
            return carry
        lax.fori_loop(0, nsb, sb_step, 0)
        plsc.subcore_barrier()

        for t in range(5):
            rs = pl.ds(s * RT + t * RC, RC)
            pltpu.sync_copy(acc_sh.at[rs], a_v)
            pltpu.sync_copy(a_v, q_out.at[c, rs])

    return k


_fspmm = _make_sc_spmm(FPAD, H, F)
_pspmm = _make_sc_spmm(EPAD, LP, N)


_R = 2000


def _dense_body(p_ref, b1_ref, w2_ref, b2_ref, out_ref):
    l1 = jnp.maximum(p_ref[0] + p_ref[1] + b1_ref[...], 0.0)
    out_ref[...] = (
        jnp.dot(l1, w2_ref[...], preferred_element_type=jnp.float32) + b2_ref[...]
    )


def _dense(p, b1, w2p, b2p):
    return pl.pallas_call(
        _dense_body,
        grid=(N // _R,),
        in_specs=[
            pl.BlockSpec((NC, _R, H), lambda i: (0, i, 0)),
            pl.BlockSpec((1, H), lambda i: (0, 0)),
            pl.BlockSpec((H, LP), lambda i: (0, 0)),
            pl.BlockSpec((1, LP), lambda i: (0, 0)),
        ],
        out_specs=pl.BlockSpec((_R, LP), lambda i: (i, 0)),
        out_shape=jax.ShapeDtypeStruct((N, LP), jnp.float32),
    )(p, b1, w2p, b2p)


def _comb_body(q_ref, l2_ref, out_ref):
    out_ref[...] = (1.0 - ALPHA) * (q_ref[0] + q_ref[1]) + ALPHA * l2_ref[...]


def _combine(q, l2):
    return pl.pallas_call(
        _comb_body,
        grid=(N // _R,),
        in_specs=[
            pl.BlockSpec((NC, _R, LP), lambda i: (0, i, 0)),
            pl.BlockSpec((_R, LP), lambda i: (i, 0)),
        ],
        out_specs=pl.BlockSpec((_R, LP), lambda i: (i, 0)),
        out_shape=jax.ShapeDtypeStruct((N, LP), jnp.float32),
    )(q, l2)


def _lsm_body(q_ref, l2_ref, out_ref):
    x = (1.0 - ALPHA) * (q_ref[0] + q_ref[1]) + ALPHA * l2_ref[...]
    x = x[:, :L]
    m = jnp.max(x, axis=1, keepdims=True)
    e = jnp.exp(x - m)
    lse = jnp.log(jnp.sum(e, axis=1, keepdims=True)) + m
    out_ref[...] = x - lse


def _lsm(q, l2):
    return pl.pallas_call(
        _lsm_body,
        grid=(N // _R,),
        in_specs=[
            pl.BlockSpec((NC, _R, LP), lambda i: (0, i, 0)),
            pl.BlockSpec((_R, LP), lambda i: (i, 0)),
        ],
        out_specs=pl.BlockSpec((_R, L), lambda i: (i, 0)),
        out_shape=jax.ShapeDtypeStruct((N, L), jnp.float32),
    )(q, l2)


def _pad2d(x, m, dtype):
    return jnp.concatenate([x, jnp.zeros((m - x.shape[0],), dtype)]).reshape(-1, 128)


def kernel(feature_indices, feature_values, edge_indices, edge_weights, W1, b1, W2, b2):
    f_src = _pad2d(feature_indices[1], FPAD, jnp.int32)
    f_dst = _pad2d(feature_indices[0], FPAD, jnp.int32)
    f_w = _pad2d(feature_values, FPAD, jnp.float32)
    e_src = _pad2d(edge_indices[1], EPAD, jnp.int32)
    e_dst = _pad2d(edge_indices[0], EPAD, jnp.int32)
    e_w = _pad2d(edge_weights, EPAD, jnp.float32)

    p = _fspmm(f_src, f_dst, f_w, W1)

    w2p = jnp.pad(W2, ((0, 0), (0, LP - L)))
    b2p = jnp.pad(b2, (0, LP - L)).reshape(1, LP)
    l2 = _dense(p, b1.reshape(1, H), w2p, b2p)

    loc = l2
    for i in range(ITERS):
        q = _pspmm(e_src, e_dst, e_w, loc)
        if i + 1 < ITERS:
            loc = _combine(q, l2)
    return _lsm(q, l2)
